# baseline (speedup 1.0000x reference)
import jax
import jax.numpy as jnp
from jax import lax
from jax.experimental import pallas as pl
from jax.experimental.pallas import tpu as pltpu
from jax.experimental.pallas import tpu_sc as plsc

B = 16384
D = 16
L = 16
NC = 2
NS = 16
NW = NC * NS
HALF = B // 2
BW = HALF // NW
R = 12
WIN = 128
SLOT_BYTES = 4 * D * WIN * 4


def _sc_body(gid_hbm, jid_hbm, ge1_hbm, je1_hbm, ge2_hbm, je2_hbm,
             gb1_hbm, jb1_hbm, gb2_hbm, jb2_hbm, wa_hbm, wb_hbm, cv_hbm,
             dummy_hbm, out_hbm,
             idx_g, idx_j, w_all, bg1, bj1, bg2, bj2,
             wa_v, wb_v, cv_v, scores_v, out_v, sems, bsem):
  wid = lax.axis_index("s") * NC + lax.axis_index("c")
  base = wid * BW

  pltpu.sync_copy(gid_hbm.at[pl.ds(base, BW)], idx_g.at[pl.ds(0, BW)])
  pltpu.sync_copy(jid_hbm.at[pl.ds(base, BW)], idx_j.at[pl.ds(0, BW)])
  pltpu.sync_copy(wa_hbm, wa_v)
  pltpu.sync_copy(wb_hbm, wb_v)
  pltpu.sync_copy(cv_hbm, cv_v)

  bcopies = [
      pltpu.async_copy(gb1_hbm.at[idx_g.at[pl.ds(0, BW)]], bg1, bsem),
      pltpu.async_copy(jb1_hbm.at[idx_j.at[pl.ds(0, BW)]], bj1, bsem),
      pltpu.async_copy(gb2_hbm.at[idx_g.at[pl.ds(0, BW)]], bg2, bsem),
      pltpu.async_copy(jb2_hbm.at[idx_j.at[pl.ds(0, BW)]], bj2, bsem),
  ]

  wav = wa_v[...]
  wbv = wb_v[...]
  dvec = lax.iota(jnp.int32, L)
  lane0 = dvec == 0

  def sread(ref, e):
    return plsc.load_gather(ref, [jnp.full((L,), e, jnp.int32)])[0]

  def fire(g, j, s):
    gcb = pl.multiple_of((g // WIN) * WIN, WIN)
    jcb = pl.multiple_of((j // WIN) * WIN, WIN)
    sem = sems.at[s]
    pltpu.async_copy(ge1_hbm.at[:, pl.ds(gcb, WIN)],
                     w_all.at[s, pl.ds(0 * D, D), :], sem)
    pltpu.async_copy(je1_hbm.at[:, pl.ds(jcb, WIN)],
                     w_all.at[s, pl.ds(1 * D, D), :], sem)
    pltpu.async_copy(ge2_hbm.at[:, pl.ds(gcb, WIN)],
                     w_all.at[s, pl.ds(2 * D, D), :], sem)
    pltpu.async_copy(je2_hbm.at[:, pl.ds(jcb, WIN)],
                     w_all.at[s, pl.ds(3 * D, D), :], sem)

  carry0 = []
  for s in range(R):
    g = sread(idx_g, s)
    j = sread(idx_j, s)
    fire(g, j, s)
    carry0 += [g, j]

  def rounds(rr, carry):
    c = list(carry)
    ebase = rr * R
    for s in range(R):
      e = ebase + s
      g = c[2 * s]
      j = c[2 * s + 1]

      @pl.when(e < BW)
      def _():
        pltpu.make_async_copy(dummy_hbm, w_all.at[s], sems.at[s]).wait()
        gl = jnp.full((L,), g % WIN, jnp.int32)
        jl = jnp.full((L,), j % WIN, jnp.int32)
        win = w_all.at[s]
        v1g = plsc.load_gather(win, [dvec, gl])
        v1j = plsc.load_gather(win, [dvec + D, jl])
        v2g = plsc.load_gather(win, [dvec + 2 * D, gl])
        v2j = plsc.load_gather(win, [dvec + 3 * D, jl])
        p = (v1g * v1j) * wav + (v2g * v2j) * wbv
        sc = jnp.sum(p)
        plsc.store_compressed(scores_v.at[pl.ds(e, L)],
                              jnp.full((L,), sc, jnp.float32), mask=lane0)
        nxt = e + R

        @pl.when(nxt < BW)
        def _():
          fire(sread(idx_g, nxt), sread(idx_j, nxt), s)

      c[2 * s] = sread(idx_g, e + R)
      c[2 * s + 1] = sread(idx_j, e + R)
    return tuple(c)

  lax.fori_loop(0, (BW + R - 1) // R, rounds, tuple(carry0))

  for c in bcopies:
    c.wait()

  def tile(t, carry):
    tb = t * L
    out_v[pl.ds(tb, L)] = (scores_v[pl.ds(tb, L)] + cv_v[...]
                           + bg1[pl.ds(tb, L)] + bj1[pl.ds(tb, L)]
                           + bg2[pl.ds(tb, L)] + bj2[pl.ds(tb, L)])
    return carry

  lax.fori_loop(0, BW // L, tile, 0)

  pltpu.sync_copy(out_v, out_hbm.at[pl.ds(base, BW)])


def kernel(geek_id, job_id, geek_emb1, job_emb1, geek_emb2, job_emb2,
           geek_b1, job_b1, geek_b2, job_b2, W1, b1, W2, b2, miu1, miu2):
  w2 = W2[:, 0]
  wa = (W1[:D] @ w2).astype(jnp.float32)
  wb = (W1[D:] @ w2).astype(jnp.float32)
  const = b1 @ w2 + b2[0] + miu1 + miu2
  cvec = jnp.full((L,), const, jnp.float32)

  gid = geek_id.astype(jnp.int32)
  jid = job_id.astype(jnp.int32)

  mesh = plsc.VectorSubcoreMesh(core_axis_name="c", subcore_axis_name="s",
                                num_cores=NC, num_subcores=NS)
  run = pl.kernel(
      _sc_body,
      out_type=jax.ShapeDtypeStruct((HALF,), jnp.float32),
      mesh=mesh,
      compiler_params=pltpu.CompilerParams(needs_layout_passes=False),
      scratch_types=[
          pltpu.VMEM((BW + 2 * R,), jnp.int32),
          pltpu.VMEM((BW + 2 * R,), jnp.int32),
          pltpu.VMEM((R, 4 * D, WIN), jnp.float32),
          pltpu.VMEM((BW,), jnp.float32),
          pltpu.VMEM((BW,), jnp.float32),
          pltpu.VMEM((BW,), jnp.float32),
          pltpu.VMEM((BW,), jnp.float32),
          pltpu.VMEM((L,), jnp.float32),
          pltpu.VMEM((L,), jnp.float32),
          pltpu.VMEM((L,), jnp.float32),
          pltpu.VMEM((BW + L,), jnp.float32),
          pltpu.VMEM((BW,), jnp.float32),
          pltpu.SemaphoreType.DMA((R,)),
          pltpu.SemaphoreType.DMA,
      ],
  )
  dummy = jnp.zeros((4 * D, WIN), jnp.float32)
  tables = (geek_emb1.T, job_emb1.T, geek_emb2.T, job_emb2.T,
            geek_b1[:, 0], job_b1[:, 0], geek_b2[:, 0], job_b2[:, 0])
  o1 = run(gid[:HALF], jid[:HALF], *tables, wa, wb, cvec, dummy)
  o2 = run(gid[HALF:], jid[HALF:], *tables, wa, wb, cvec, dummy)
  return jnp.concatenate([o1, o2])


# speedup vs baseline: 1.0357x; 1.0357x over previous
import jax
import jax.numpy as jnp
from jax import lax
from jax.experimental import pallas as pl
from jax.experimental.pallas import tpu as pltpu
from jax.experimental.pallas import tpu_sc as plsc

B = 16384
D = 16
L = 16
NC = 2
NS = 16
NW = NC * NS
BW = B // NW
R = 12
WIN = 128
SLOT_BYTES = 4 * D * WIN * 4


def _sc_body(gid_hbm, jid_hbm, ge1_hbm, je1_hbm, ge2_hbm, je2_hbm,
             gb1_hbm, jb1_hbm, gb2_hbm, jb2_hbm, wa_hbm, wb_hbm, cv_hbm,
             dummy_hbm, out_hbm,
             idx_g, idx_j, w_all, bg1, bj1, bg2, bj2,
             wa_v, wb_v, cv_v, scores_v, out_v, sems, bsem):
  wid = lax.axis_index("s") * NC + lax.axis_index("c")
  base = wid * BW

  pltpu.sync_copy(gid_hbm.at[pl.ds(base, BW)], idx_g.at[pl.ds(0, BW)])
  pltpu.sync_copy(jid_hbm.at[pl.ds(base, BW)], idx_j.at[pl.ds(0, BW)])
  pltpu.sync_copy(wa_hbm, wa_v)
  pltpu.sync_copy(wb_hbm, wb_v)
  pltpu.sync_copy(cv_hbm, cv_v)

  bcopies = [
      pltpu.async_copy(gb1_hbm.at[idx_g.at[pl.ds(0, BW)]], bg1, bsem),
      pltpu.async_copy(jb1_hbm.at[idx_j.at[pl.ds(0, BW)]], bj1, bsem),
      pltpu.async_copy(gb2_hbm.at[idx_g.at[pl.ds(0, BW)]], bg2, bsem),
      pltpu.async_copy(jb2_hbm.at[idx_j.at[pl.ds(0, BW)]], bj2, bsem),
  ]

  wav = wa_v[...]
  wbv = wb_v[...]
  dvec = lax.iota(jnp.int32, L)
  lane0 = dvec == 0

  def sread(ref, e):
    return plsc.load_gather(ref, [jnp.full((L,), e, jnp.int32)])[0]

  def fire(g, j, s):
    gcb = pl.multiple_of((g // WIN) * WIN, WIN)
    jcb = pl.multiple_of((j // WIN) * WIN, WIN)
    sem = sems.at[s]
    pltpu.async_copy(ge1_hbm.at[:, pl.ds(gcb, WIN)],
                     w_all.at[s, pl.ds(0 * D, D), :], sem)
    pltpu.async_copy(je1_hbm.at[:, pl.ds(jcb, WIN)],
                     w_all.at[s, pl.ds(1 * D, D), :], sem)
    pltpu.async_copy(ge2_hbm.at[:, pl.ds(gcb, WIN)],
                     w_all.at[s, pl.ds(2 * D, D), :], sem)
    pltpu.async_copy(je2_hbm.at[:, pl.ds(jcb, WIN)],
                     w_all.at[s, pl.ds(3 * D, D), :], sem)

  carry0 = []
  for s in range(R):
    g = sread(idx_g, s)
    j = sread(idx_j, s)
    fire(g, j, s)
    carry0 += [g, j]

  def rounds(rr, carry):
    c = list(carry)
    ebase = rr * R
    for s in range(R):
      e = ebase + s
      g = c[2 * s]
      j = c[2 * s + 1]

      @pl.when(e < BW)
      def _():
        pltpu.make_async_copy(dummy_hbm, w_all.at[s], sems.at[s]).wait()
        gl = jnp.full((L,), g % WIN, jnp.int32)
        jl = jnp.full((L,), j % WIN, jnp.int32)
        win = w_all.at[s]
        v1g = plsc.load_gather(win, [dvec, gl])
        v1j = plsc.load_gather(win, [dvec + D, jl])
        v2g = plsc.load_gather(win, [dvec + 2 * D, gl])
        v2j = plsc.load_gather(win, [dvec + 3 * D, jl])
        p = (v1g * v1j) * wav + (v2g * v2j) * wbv
        sc = jnp.sum(p)
        plsc.store_compressed(scores_v.at[pl.ds(e, L)],
                              jnp.full((L,), sc, jnp.float32), mask=lane0)
        nxt = e + R

        @pl.when(nxt < BW)
        def _():
          fire(sread(idx_g, nxt), sread(idx_j, nxt), s)

      c[2 * s] = sread(idx_g, e + R)
      c[2 * s + 1] = sread(idx_j, e + R)
    return tuple(c)

  lax.fori_loop(0, (BW + R - 1) // R, rounds, tuple(carry0))

  for c in bcopies:
    c.wait()

  def tile(t, carry):
    tb = t * L
    out_v[pl.ds(tb, L)] = (scores_v[pl.ds(tb, L)] + cv_v[...]
                           + bg1[pl.ds(tb, L)] + bj1[pl.ds(tb, L)]
                           + bg2[pl.ds(tb, L)] + bj2[pl.ds(tb, L)])
    return carry

  lax.fori_loop(0, BW // L, tile, 0)

  pltpu.sync_copy(out_v, out_hbm.at[pl.ds(base, BW)])


def kernel(geek_id, job_id, geek_emb1, job_emb1, geek_emb2, job_emb2,
           geek_b1, job_b1, geek_b2, job_b2, W1, b1, W2, b2, miu1, miu2):
  w2 = W2[:, 0]
  wa = (W1[:D] @ w2).astype(jnp.float32)
  wb = (W1[D:] @ w2).astype(jnp.float32)
  const = b1 @ w2 + b2[0] + miu1 + miu2
  cvec = jnp.full((L,), const, jnp.float32)

  gid = geek_id.astype(jnp.int32)
  jid = job_id.astype(jnp.int32)

  mesh = plsc.VectorSubcoreMesh(core_axis_name="c", subcore_axis_name="s",
                                num_cores=NC, num_subcores=NS)
  run = pl.kernel(
      _sc_body,
      out_type=jax.ShapeDtypeStruct((B,), jnp.float32),
      mesh=mesh,
      compiler_params=pltpu.CompilerParams(needs_layout_passes=False),
      scratch_types=[
          pltpu.VMEM((BW + 2 * R,), jnp.int32),
          pltpu.VMEM((BW + 2 * R,), jnp.int32),
          pltpu.VMEM((R, 4 * D, WIN), jnp.float32),
          pltpu.VMEM((BW,), jnp.float32),
          pltpu.VMEM((BW,), jnp.float32),
          pltpu.VMEM((BW,), jnp.float32),
          pltpu.VMEM((BW,), jnp.float32),
          pltpu.VMEM((L,), jnp.float32),
          pltpu.VMEM((L,), jnp.float32),
          pltpu.VMEM((L,), jnp.float32),
          pltpu.VMEM((BW + L,), jnp.float32),
          pltpu.VMEM((BW,), jnp.float32),
          pltpu.SemaphoreType.DMA((R,)),
          pltpu.SemaphoreType.DMA,
      ],
  )
  dummy = jnp.zeros((4 * D, WIN), jnp.float32)
  return run(gid, jid, geek_emb1.T, job_emb1.T, geek_emb2.T, job_emb2.T,
             geek_b1[:, 0], job_b1[:, 0], geek_b2[:, 0], job_b2[:, 0],
             wa, wb, cvec, dummy)
